# baseline (speedup 1.0000x reference)
import functools

import jax
import jax.numpy as jnp
from jax import lax
from jax.experimental import pallas as pl
from jax.experimental.pallas import tpu as pltpu
from jax.experimental.pallas import tpu_sc as plsc

NC = 2
NS = 16
L = 16
NW = NC * NS

IDX_CHUNK = 128


@functools.lru_cache(maxsize=None)
def _build_sc_kernel(B: int, D: int):
    assert B % (NW * L) == 0
    BPW = B // NW
    NCH = BPW // IDX_CHUNK
    CH = BPW // L

    mesh = plsc.VectorSubcoreMesh(core_axis_name="c", subcore_axis_name="s")

    @functools.partial(
        pl.kernel,
        out_type=jax.ShapeDtypeStruct((B,), jnp.float32),
        mesh=mesh,
        scratch_types=[
            pltpu.VMEM((NCH, IDX_CHUNK), jnp.int32),
            pltpu.VMEM((NCH, IDX_CHUNK), jnp.int32),
            pltpu.VMEM((BPW, D), jnp.float32),
            pltpu.VMEM((BPW, D), jnp.float32),
            pltpu.VMEM((BPW,), jnp.float32),
            pltpu.VMEM((BPW,), jnp.float32),
            pltpu.VMEM((L * L,), jnp.float32),
            pltpu.VMEM((BPW,), jnp.float32),
            pltpu.SemaphoreType.DMA,
        ],
        compiler_params=pltpu.CompilerParams(needs_layout_passes=False,
                                             use_tc_tiling_on_sc=False),
    )
    def sc_kernel(uidx_hbm, iidx_hbm, uf_hbm, if_hbm, ub_hbm, ib_hbm,
                  out_hbm, uidx_v, iidx_v, urows_v, irows_v, ub_v, ib_v,
                  prod_v, out_v, sem):
        wid = lax.axis_index("s") * NC + lax.axis_index("c")
        base = wid * BPW

        pltpu.sync_copy(uidx_hbm.at[pl.ds(wid * NCH, NCH)], uidx_v)
        pltpu.sync_copy(iidx_hbm.at[pl.ds(wid * NCH, NCH)], iidx_v)

        iota = lax.iota(jnp.int32, L)

        copies = []
        for j in range(NCH):
            dst = pl.ds(j * IDX_CHUNK, IDX_CHUNK)
            copies.append(pltpu.make_async_copy(
                uf_hbm.at[uidx_v.at[j]], urows_v.at[dst], sem))
            copies.append(pltpu.make_async_copy(
                if_hbm.at[iidx_v.at[j]], irows_v.at[dst], sem))
            copies.append(pltpu.make_async_copy(
                ub_hbm.at[uidx_v.at[j]], ub_v.at[dst], sem))
            copies.append(pltpu.make_async_copy(
                ib_hbm.at[iidx_v.at[j]], ib_v.at[dst], sem))
        for c in copies:
            c.start()
        for c in copies:
            c.wait()

        nh = D // L
        def comp_body(i, carry):
            rbase = i * L
            for r in range(L):
                t = jnp.zeros((L,), jnp.float32)
                for h in range(nh):
                    t += (urows_v[rbase + r, pl.ds(h * L, L)]
                          * irows_v[rbase + r, pl.ds(h * L, L)])
                plsc.store_scatter(prod_v, [iota * L + r], t)
            acc = ub_v[pl.ds(rbase, L)] + ib_v[pl.ds(rbase, L)]
            for d in range(L):
                acc += prod_v[pl.ds(d * L, L)]
            out_v[pl.ds(rbase, L)] = acc
            return carry
        lax.fori_loop(0, CH, comp_body, 0)

        pltpu.sync_copy(out_v, out_hbm.at[pl.ds(base, BPW)])

    return sc_kernel


def kernel(user_item_tuple, user_factors, item_factors, user_biases,
           item_biases):
    uit = user_item_tuple.astype(jnp.int32)
    B = uit.shape[0]
    D = user_factors.shape[1]
    u_idx = uit[:, 0].reshape(B // IDX_CHUNK, IDX_CHUNK)
    i_idx = uit[:, 1].reshape(B // IDX_CHUNK, IDX_CHUNK)
    return _build_sc_kernel(B, D)(u_idx, i_idx, user_factors, item_factors,
                                  user_biases.reshape(-1),
                                  item_biases.reshape(-1))
